# Initial kernel scaffold; baseline (speedup 1.0000x reference)
import jax
import jax.numpy as jnp
from jax.experimental import pallas as pl


def kernel(code, numeric_value, time_delta_days, static_mask, numeric_value_mask, mask, code_table, date_w, date_b, val_w, val_b):
    raise NotImplementedError("write your pallas kernel here")


# speedup vs baseline: 1.1382x; 1.1382x over previous
import functools

import jax
import jax.numpy as jnp
from jax import lax
from jax.experimental import pallas as pl
from jax.experimental.pallas import tpu as pltpu
from jax.experimental.pallas import tpu_sc as plsc

_B, _T, _D, _V = 1024, 200, 128, 100000
_BT = _B * _T

_NC, _NS = 2, 16
_NW = _NC * _NS
_CH = 128
_ROWS_PER_W = _BT // _NW
_CPW = _ROWS_PER_W // _CH
_SPW = 56


def _sc_gather_body(code_hbm, table_hbm, out_hbm, idx_v, buf_v, sem0, sem1):
    c = lax.axis_index("c")
    s = lax.axis_index("s")
    w = s * _NC + c
    ibase = w * _SPW
    rbase = w * _ROWS_PER_W

    pltpu.sync_copy(code_hbm.at[pl.ds(ibase, _SPW)], idx_v)

    pltpu.async_copy(table_hbm.at[idx_v.at[0]], buf_v.at[0], sem0)

    def body(i, carry):
        j0 = 2 * i
        j1 = j0 + 1
        pltpu.async_copy(table_hbm.at[idx_v.at[j1]], buf_v.at[1], sem1)
        pltpu.make_async_copy(table_hbm.at[idx_v.at[j0]], buf_v.at[0], sem0).wait()
        pltpu.sync_copy(buf_v.at[0], out_hbm.at[pl.ds(rbase + j0 * _CH, _CH)])

        @pl.when(j1 + 1 < _CPW)
        def _():
            pltpu.async_copy(table_hbm.at[idx_v.at[j1 + 1]], buf_v.at[0], sem0)

        pltpu.make_async_copy(table_hbm.at[idx_v.at[j1]], buf_v.at[1], sem1).wait()
        pltpu.sync_copy(buf_v.at[1], out_hbm.at[pl.ds(rbase + j1 * _CH, _CH)])
        return carry

    lax.fori_loop(0, _CPW // 2, body, 0)


@functools.lru_cache(maxsize=1)
def _sc_gather_fn():
    return pl.kernel(
        _sc_gather_body,
        out_type=jax.ShapeDtypeStruct((_BT, _D), jnp.float32),
        mesh=plsc.VectorSubcoreMesh(core_axis_name="c", subcore_axis_name="s",
                                    num_cores=_NC, num_subcores=_NS),
        scratch_types=[
            pltpu.VMEM((_SPW, _CH), jnp.int32),
            pltpu.VMEM((2, _CH, _D), jnp.float32),
            pltpu.SemaphoreType.DMA,
            pltpu.SemaphoreType.DMA,
        ],
    )


def _sc_gather(code2, table):
    return _sc_gather_fn()(code2, table)


_BB = 8


def _tc_body(g, td, nv, sm, nm, dw, db, vw, vb, emb_o, te_o, ce_o, ve_o):
    rows = g[...]
    ce = jnp.transpose(rows, (0, 2, 1))
    dwc = jnp.transpose(dw[...], (1, 0))
    dbc = jnp.transpose(db[...], (1, 0))
    vwc = jnp.transpose(vw[...], (1, 0))
    vbc = jnp.transpose(vb[...], (1, 0))
    td3 = td[...][:, None, :]
    nv3 = nv[...][:, None, :]
    sm3 = sm[...][:, None, :]
    nm3 = nm[...][:, None, :]
    te = (td3 * dwc[None] + dbc[None]) * sm3
    ve = (nv3 * vwc[None] + vbc[None]) * nm3
    emb_o[...] = te + ce + ve
    te_o[...] = te
    ce_o[...] = ce
    ve_o[...] = ve


def _tc_call(g3, td, nv, sm, nm, dw2, db2, vw2, vb2):
    bt_spec = pl.BlockSpec((_BB, _T), lambda i: (i, 0))
    w_spec = pl.BlockSpec((1, _D), lambda i: (0, 0))
    out_spec = pl.BlockSpec((_BB, _D, _T), lambda i: (i, 0, 0))
    out_shape = jax.ShapeDtypeStruct((_B, _D, _T), jnp.float32)
    return pl.pallas_call(
        _tc_body,
        grid=(_B // _BB,),
        in_specs=[
            pl.BlockSpec((_BB, _T, _D), lambda i: (i, 0, 0)),
            bt_spec, bt_spec, bt_spec, bt_spec,
            w_spec, w_spec, w_spec, w_spec,
        ],
        out_specs=[out_spec, out_spec, out_spec, out_spec],
        out_shape=[out_shape, out_shape, out_shape, out_shape],
    )(g3, td, nv, sm, nm, dw2, db2, vw2, vb2)


def kernel(code, numeric_value, time_delta_days, static_mask, numeric_value_mask,
           mask, code_table, date_w, date_b, val_w, val_b):
    code2 = code.astype(jnp.int32).reshape(_BT // _CH, _CH)
    code2 = code2.reshape(_NW, _CPW, _CH)
    code2 = jnp.pad(code2, ((0, 0), (0, _SPW - _CPW), (0, 0)))
    code2 = code2.reshape(_NW * _SPW, _CH)
    gathered = _sc_gather(code2, code_table)
    g3 = gathered.reshape(_B, _T, _D)
    sm = (~static_mask).astype(jnp.float32)
    nm = numeric_value_mask.astype(jnp.float32)
    emb, te, ce, ve = _tc_call(
        g3, time_delta_days, numeric_value, sm, nm,
        date_w.reshape(1, _D), date_b.reshape(1, _D),
        val_w.reshape(1, _D), val_b.reshape(1, _D),
    )
    return (emb, te, ce, ve)


# speedup vs baseline: 2.9544x; 2.5956x over previous
import functools

import jax
import jax.numpy as jnp
from jax import lax
from jax.experimental import pallas as pl
from jax.experimental.pallas import tpu as pltpu
from jax.experimental.pallas import tpu_sc as plsc

_B, _T, _D, _V = 1024, 200, 128, 100000
_BT = _B * _T

_NC, _NS = 2, 16
_NW = _NC * _NS
_CH = 128
_ROWS_PER_W = _BT // _NW
_CPW = _ROWS_PER_W // _CH
_SPW = 56


def _sc_gather_body(code_hbm, table_hbm, out_hbm, idx_v, buf_v, sem0, sem1):
    c = lax.axis_index("c")
    s = lax.axis_index("s")
    w = s * _NC + c
    ibase = w * _SPW
    rbase = w * _ROWS_PER_W

    pltpu.sync_copy(code_hbm.at[pl.ds(ibase, _SPW)], idx_v)

    pltpu.async_copy(table_hbm.at[idx_v.at[0]], buf_v.at[0], sem0)

    def body(i, carry):
        j0 = 2 * i
        j1 = j0 + 1
        pltpu.async_copy(table_hbm.at[idx_v.at[j1]], buf_v.at[1], sem1)
        pltpu.make_async_copy(table_hbm.at[idx_v.at[j0]], buf_v.at[0], sem0).wait()
        pltpu.sync_copy(buf_v.at[0], out_hbm.at[pl.ds(rbase + j0 * _CH, _CH)])

        @pl.when(j1 + 1 < _CPW)
        def _():
            pltpu.async_copy(table_hbm.at[idx_v.at[j1 + 1]], buf_v.at[0], sem0)

        pltpu.make_async_copy(table_hbm.at[idx_v.at[j1]], buf_v.at[1], sem1).wait()
        pltpu.sync_copy(buf_v.at[1], out_hbm.at[pl.ds(rbase + j1 * _CH, _CH)])
        return carry

    lax.fori_loop(0, _CPW // 2, body, 0)


@functools.lru_cache(maxsize=1)
def _sc_gather_fn():
    return pl.kernel(
        _sc_gather_body,
        out_type=jax.ShapeDtypeStruct((_BT, _D), jnp.float32),
        mesh=plsc.VectorSubcoreMesh(core_axis_name="c", subcore_axis_name="s",
                                    num_cores=_NC, num_subcores=_NS),
        scratch_types=[
            pltpu.VMEM((_SPW, _CH), jnp.int32),
            pltpu.VMEM((2, _CH, _D), jnp.float32),
            pltpu.SemaphoreType.DMA,
            pltpu.SemaphoreType.DMA,
        ],
    )


def _sc_gather(code2, table):
    return _sc_gather_fn()(code2, table)


_BB = 8


def _tc_body(g, td, nv, sm, nm, dw, db, vw, vb, emb_o, te_o, ve_o):
    rows = g[...]
    te = (td[...][:, :, None] * dw[...][None] + db[...][None]) * sm[...][:, :, None]
    ve = (nv[...][:, :, None] * vw[...][None] + vb[...][None]) * nm[...][:, :, None]
    emb_o[...] = te + rows + ve
    te_o[...] = te
    ve_o[...] = ve


def _tc_call(g3, td, nv, sm, nm, dw2, db2, vw2, vb2):
    bt_spec = pl.BlockSpec((_BB, _T), lambda i: (i, 0))
    w_spec = pl.BlockSpec((1, _D), lambda i: (0, 0))
    out_spec = pl.BlockSpec((_BB, _T, _D), lambda i: (i, 0, 0))
    out_shape = jax.ShapeDtypeStruct((_B, _T, _D), jnp.float32)
    return pl.pallas_call(
        _tc_body,
        grid=(_B // _BB,),
        in_specs=[
            pl.BlockSpec((_BB, _T, _D), lambda i: (i, 0, 0)),
            bt_spec, bt_spec, bt_spec, bt_spec,
            w_spec, w_spec, w_spec, w_spec,
        ],
        out_specs=[out_spec, out_spec, out_spec],
        out_shape=[out_shape, out_shape, out_shape],
    )(g3, td, nv, sm, nm, dw2, db2, vw2, vb2)


def kernel(code, numeric_value, time_delta_days, static_mask, numeric_value_mask,
           mask, code_table, date_w, date_b, val_w, val_b):
    code2 = code.astype(jnp.int32).reshape(_BT // _CH, _CH)
    code2 = code2.reshape(_NW, _CPW, _CH)
    code2 = jnp.pad(code2, ((0, 0), (0, _SPW - _CPW), (0, 0)))
    code2 = code2.reshape(_NW * _SPW, _CH)
    gathered = _sc_gather(code2, code_table)
    g3 = gathered.reshape(_B, _T, _D)
    sm = (~static_mask).astype(jnp.float32)
    nm = numeric_value_mask.astype(jnp.float32)
    emb, te, ve = _tc_call(
        g3, time_delta_days, numeric_value, sm, nm,
        date_w.reshape(1, _D), date_b.reshape(1, _D),
        val_w.reshape(1, _D), val_b.reshape(1, _D),
    )
    tr = lambda x: jnp.transpose(x, (0, 2, 1))
    return (tr(emb), tr(te), tr(g3), tr(ve))


# speedup vs baseline: 3.5018x; 1.1853x over previous
import functools

import jax
import jax.numpy as jnp
from jax import lax
from jax.experimental import pallas as pl
from jax.experimental.pallas import tpu as pltpu
from jax.experimental.pallas import tpu_sc as plsc

_B, _T, _D, _V = 1024, 200, 128, 100000
_BT = _B * _T

_NC, _NS = 2, 16
_NW = _NC * _NS
_CH = 128
_ROWS_PER_W = _BT // _NW
_CPW = _ROWS_PER_W // _CH
_SPW = 56


def _sc_gather_body(code_hbm, table_hbm, out_hbm, idx_v, buf_v, sem0, sem1):
    c = lax.axis_index("c")
    s = lax.axis_index("s")
    w = s * _NC + c
    ibase = w * _SPW
    rbase = w * _ROWS_PER_W

    pltpu.sync_copy(code_hbm.at[pl.ds(ibase, _SPW)], idx_v)

    pltpu.async_copy(table_hbm.at[idx_v.at[0]], buf_v.at[0], sem0)

    def body(i, carry):
        j0 = 2 * i
        j1 = j0 + 1
        pltpu.async_copy(table_hbm.at[idx_v.at[j1]], buf_v.at[1], sem1)
        pltpu.make_async_copy(table_hbm.at[idx_v.at[j0]], buf_v.at[0], sem0).wait()
        pltpu.sync_copy(buf_v.at[0], out_hbm.at[pl.ds(rbase + j0 * _CH, _CH)])

        @pl.when(j1 + 1 < _CPW)
        def _():
            pltpu.async_copy(table_hbm.at[idx_v.at[j1 + 1]], buf_v.at[0], sem0)

        pltpu.make_async_copy(table_hbm.at[idx_v.at[j1]], buf_v.at[1], sem1).wait()
        pltpu.sync_copy(buf_v.at[1], out_hbm.at[pl.ds(rbase + j1 * _CH, _CH)])
        return carry

    lax.fori_loop(0, _CPW // 2, body, 0)


@functools.lru_cache(maxsize=1)
def _sc_gather_fn():
    return pl.kernel(
        _sc_gather_body,
        out_type=jax.ShapeDtypeStruct((_BT, _D), jnp.float32),
        mesh=plsc.VectorSubcoreMesh(core_axis_name="c", subcore_axis_name="s",
                                    num_cores=_NC, num_subcores=_NS),
        scratch_types=[
            pltpu.VMEM((_SPW, _CH), jnp.int32),
            pltpu.VMEM((2, _CH, _D), jnp.float32),
            pltpu.SemaphoreType.DMA,
            pltpu.SemaphoreType.DMA,
        ],
    )


def _sc_gather(code2, table):
    return _sc_gather_fn()(code2, table)


_BB = 16


def _tc_body(g, td, nv, sm, nm, dw, db, vw, vb, emb_o, te_o, ve_o):
    rows = g[...]
    te = (td[...][:, :, None] * dw[...][None] + db[...][None]) * sm[...][:, :, None]
    ve = (nv[...][:, :, None] * vw[...][None] + vb[...][None]) * nm[...][:, :, None]
    emb_o[...] = te + rows + ve
    te_o[...] = te
    ve_o[...] = ve


def _tc_call(g3, td, nv, sm, nm, dw2, db2, vw2, vb2):
    bt_spec = pl.BlockSpec((_BB, _T), lambda i: (i, 0))
    w_spec = pl.BlockSpec((1, _D), lambda i: (0, 0))
    out_spec = pl.BlockSpec((_BB, _T, _D), lambda i: (i, 0, 0))
    out_shape = jax.ShapeDtypeStruct((_B, _T, _D), jnp.float32)
    return pl.pallas_call(
        _tc_body,
        grid=(_B // _BB,),
        in_specs=[
            pl.BlockSpec((_BB, _T, _D), lambda i: (i, 0, 0)),
            bt_spec, bt_spec, bt_spec, bt_spec,
            w_spec, w_spec, w_spec, w_spec,
        ],
        out_specs=[out_spec, out_spec, out_spec],
        out_shape=[out_shape, out_shape, out_shape],
    )(g3, td, nv, sm, nm, dw2, db2, vw2, vb2)


def kernel(code, numeric_value, time_delta_days, static_mask, numeric_value_mask,
           mask, code_table, date_w, date_b, val_w, val_b):
    code2 = code.astype(jnp.int32).reshape(_BT // _CH, _CH)
    code2 = code2.reshape(_NW, _CPW, _CH)
    code2 = jnp.pad(code2, ((0, 0), (0, _SPW - _CPW), (0, 0)))
    code2 = code2.reshape(_NW * _SPW, _CH)
    gathered = _sc_gather(code2, code_table)
    g3 = gathered.reshape(_B, _T, _D)
    sm = (~static_mask).astype(jnp.float32)
    nm = numeric_value_mask.astype(jnp.float32)
    emb, te, ve = _tc_call(
        g3, time_delta_days, numeric_value, sm, nm,
        date_w.reshape(1, _D), date_b.reshape(1, _D),
        val_w.reshape(1, _D), val_b.reshape(1, _D),
    )
    tr = lambda x: jnp.transpose(x, (0, 2, 1))
    return (tr(emb), tr(te), tr(g3), tr(ve))


# speedup vs baseline: 3.7550x; 1.0723x over previous
import functools

import jax
import jax.numpy as jnp
from jax import lax
from jax.experimental import pallas as pl
from jax.experimental.pallas import tpu as pltpu
from jax.experimental.pallas import tpu_sc as plsc

_B, _T, _D, _V = 1024, 200, 128, 100000
_BT = _B * _T

_NC, _NS = 2, 16
_NW = _NC * _NS
_CH = 128
_ROWS_PER_W = _BT // _NW
_CPW = _ROWS_PER_W // _CH
_SPW = 56


def _sc_gather_body(code_hbm, table_hbm, out_hbm, idx_v, buf_v, sem0, sem1):
    c = lax.axis_index("c")
    s = lax.axis_index("s")
    w = s * _NC + c
    ibase = w * _SPW
    rbase = w * _ROWS_PER_W

    pltpu.sync_copy(code_hbm.at[pl.ds(ibase, _SPW)], idx_v)

    pltpu.async_copy(table_hbm.at[idx_v.at[0]], buf_v.at[0], sem0)

    def body(i, carry):
        j0 = 2 * i
        j1 = j0 + 1
        pltpu.async_copy(table_hbm.at[idx_v.at[j1]], buf_v.at[1], sem1)
        pltpu.make_async_copy(table_hbm.at[idx_v.at[j0]], buf_v.at[0], sem0).wait()
        pltpu.sync_copy(buf_v.at[0], out_hbm.at[pl.ds(rbase + j0 * _CH, _CH)])

        @pl.when(j1 + 1 < _CPW)
        def _():
            pltpu.async_copy(table_hbm.at[idx_v.at[j1 + 1]], buf_v.at[0], sem0)

        pltpu.make_async_copy(table_hbm.at[idx_v.at[j1]], buf_v.at[1], sem1).wait()
        pltpu.sync_copy(buf_v.at[1], out_hbm.at[pl.ds(rbase + j1 * _CH, _CH)])
        return carry

    lax.fori_loop(0, _CPW // 2, body, 0)


@functools.lru_cache(maxsize=1)
def _sc_gather_fn():
    return pl.kernel(
        _sc_gather_body,
        out_type=jax.ShapeDtypeStruct((_BT, _D), jnp.float32),
        mesh=plsc.VectorSubcoreMesh(core_axis_name="c", subcore_axis_name="s",
                                    num_cores=_NC, num_subcores=_NS),
        scratch_types=[
            pltpu.VMEM((_SPW, _CH), jnp.int32),
            pltpu.VMEM((2, _CH, _D), jnp.float32),
            pltpu.SemaphoreType.DMA,
            pltpu.SemaphoreType.DMA,
        ],
    )


def _sc_gather(code2, table):
    return _sc_gather_fn()(code2, table)


_BB = 32


def _tc_body(g, td, nv, sm, nm, dw, db, vw, vb, emb_o, te_o, ve_o):
    rows = g[...]
    te = (td[...][:, :, None] * dw[...][None] + db[...][None]) * sm[...][:, :, None]
    ve = (nv[...][:, :, None] * vw[...][None] + vb[...][None]) * nm[...][:, :, None]
    emb_o[...] = te + rows + ve
    te_o[...] = te
    ve_o[...] = ve


def _tc_call(g3, td, nv, sm, nm, dw2, db2, vw2, vb2):
    bt_spec = pl.BlockSpec((_BB, _T), lambda i: (i, 0))
    w_spec = pl.BlockSpec((1, _D), lambda i: (0, 0))
    out_spec = pl.BlockSpec((_BB, _T, _D), lambda i: (i, 0, 0))
    out_shape = jax.ShapeDtypeStruct((_B, _T, _D), jnp.float32)
    return pl.pallas_call(
        _tc_body,
        grid=(_B // _BB,),
        in_specs=[
            pl.BlockSpec((_BB, _T, _D), lambda i: (i, 0, 0)),
            bt_spec, bt_spec, bt_spec, bt_spec,
            w_spec, w_spec, w_spec, w_spec,
        ],
        out_specs=[out_spec, out_spec, out_spec],
        out_shape=[out_shape, out_shape, out_shape],
    )(g3, td, nv, sm, nm, dw2, db2, vw2, vb2)


def kernel(code, numeric_value, time_delta_days, static_mask, numeric_value_mask,
           mask, code_table, date_w, date_b, val_w, val_b):
    code2 = code.astype(jnp.int32).reshape(_BT // _CH, _CH)
    code2 = code2.reshape(_NW, _CPW, _CH)
    code2 = jnp.pad(code2, ((0, 0), (0, _SPW - _CPW), (0, 0)))
    code2 = code2.reshape(_NW * _SPW, _CH)
    gathered = _sc_gather(code2, code_table)
    g3 = gathered.reshape(_B, _T, _D)
    sm = (~static_mask).astype(jnp.float32)
    nm = numeric_value_mask.astype(jnp.float32)
    emb, te, ve = _tc_call(
        g3, time_delta_days, numeric_value, sm, nm,
        date_w.reshape(1, _D), date_b.reshape(1, _D),
        val_w.reshape(1, _D), val_b.reshape(1, _D),
    )
    tr = lambda x: jnp.transpose(x, (0, 2, 1))
    return (tr(emb), tr(te), tr(g3), tr(ve))


# speedup vs baseline: 3.7886x; 1.0089x over previous
import functools

import jax
import jax.numpy as jnp
from jax import lax
from jax.experimental import pallas as pl
from jax.experimental.pallas import tpu as pltpu
from jax.experimental.pallas import tpu_sc as plsc

_B, _T, _D, _V = 1024, 200, 128, 100000
_BT = _B * _T

_NC, _NS = 2, 16
_NW = _NC * _NS
_CH = 128
_ROWS_PER_W = _BT // _NW
_CPW = _ROWS_PER_W // _CH
_SPW = 56


def _sc_gather_body(code_hbm, table_hbm, out_hbm, idx_v, buf_v, sem0, sem1):
    c = lax.axis_index("c")
    s = lax.axis_index("s")
    w = s * _NC + c
    ibase = w * _SPW
    rbase = w * _ROWS_PER_W

    pltpu.sync_copy(code_hbm.at[pl.ds(ibase, _SPW)], idx_v)

    pltpu.async_copy(table_hbm.at[idx_v.at[0]], buf_v.at[0], sem0)

    def body(i, carry):
        j0 = 2 * i
        j1 = j0 + 1
        pltpu.async_copy(table_hbm.at[idx_v.at[j1]], buf_v.at[1], sem1)
        pltpu.make_async_copy(table_hbm.at[idx_v.at[j0]], buf_v.at[0], sem0).wait()
        pltpu.sync_copy(buf_v.at[0], out_hbm.at[pl.ds(rbase + j0 * _CH, _CH)])

        @pl.when(j1 + 1 < _CPW)
        def _():
            pltpu.async_copy(table_hbm.at[idx_v.at[j1 + 1]], buf_v.at[0], sem0)

        pltpu.make_async_copy(table_hbm.at[idx_v.at[j1]], buf_v.at[1], sem1).wait()
        pltpu.sync_copy(buf_v.at[1], out_hbm.at[pl.ds(rbase + j1 * _CH, _CH)])
        return carry

    lax.fori_loop(0, _CPW // 2, body, 0)


@functools.lru_cache(maxsize=1)
def _sc_gather_fn():
    return pl.kernel(
        _sc_gather_body,
        out_type=jax.ShapeDtypeStruct((_BT, _D), jnp.float32),
        mesh=plsc.VectorSubcoreMesh(core_axis_name="c", subcore_axis_name="s",
                                    num_cores=_NC, num_subcores=_NS),
        scratch_types=[
            pltpu.VMEM((_SPW, _CH), jnp.int32),
            pltpu.VMEM((2, _CH, _D), jnp.float32),
            pltpu.SemaphoreType.DMA,
            pltpu.SemaphoreType.DMA,
        ],
    )


def _sc_gather(code2, table):
    return _sc_gather_fn()(code2, table)


_BB = 64


def _tc_body(g, td, nv, sm, nm, dw, db, vw, vb, emb_o, te_o, ve_o):
    rows = g[...]
    te = (td[...][:, :, None] * dw[...][None] + db[...][None]) * sm[...][:, :, None]
    ve = (nv[...][:, :, None] * vw[...][None] + vb[...][None]) * nm[...][:, :, None]
    emb_o[...] = te + rows + ve
    te_o[...] = te
    ve_o[...] = ve


def _tc_call(g3, td, nv, sm, nm, dw2, db2, vw2, vb2):
    bt_spec = pl.BlockSpec((_BB, _T), lambda i: (i, 0))
    w_spec = pl.BlockSpec((1, _D), lambda i: (0, 0))
    out_spec = pl.BlockSpec((_BB, _T, _D), lambda i: (i, 0, 0))
    out_shape = jax.ShapeDtypeStruct((_B, _T, _D), jnp.float32)
    return pl.pallas_call(
        _tc_body,
        grid=(_B // _BB,),
        in_specs=[
            pl.BlockSpec((_BB, _T, _D), lambda i: (i, 0, 0)),
            bt_spec, bt_spec, bt_spec, bt_spec,
            w_spec, w_spec, w_spec, w_spec,
        ],
        out_specs=[out_spec, out_spec, out_spec],
        out_shape=[out_shape, out_shape, out_shape],
    )(g3, td, nv, sm, nm, dw2, db2, vw2, vb2)


def kernel(code, numeric_value, time_delta_days, static_mask, numeric_value_mask,
           mask, code_table, date_w, date_b, val_w, val_b):
    code2 = code.astype(jnp.int32).reshape(_BT // _CH, _CH)
    code2 = code2.reshape(_NW, _CPW, _CH)
    code2 = jnp.pad(code2, ((0, 0), (0, _SPW - _CPW), (0, 0)))
    code2 = code2.reshape(_NW * _SPW, _CH)
    gathered = _sc_gather(code2, code_table)
    g3 = gathered.reshape(_B, _T, _D)
    sm = (~static_mask).astype(jnp.float32)
    nm = numeric_value_mask.astype(jnp.float32)
    emb, te, ve = _tc_call(
        g3, time_delta_days, numeric_value, sm, nm,
        date_w.reshape(1, _D), date_b.reshape(1, _D),
        val_w.reshape(1, _D), val_b.reshape(1, _D),
    )
    tr = lambda x: jnp.transpose(x, (0, 2, 1))
    return (tr(emb), tr(te), tr(g3), tr(ve))
